# baseline (speedup 1.0000x reference)
import functools

import jax
import jax.numpy as jnp
from jax import lax
from jax.experimental import pallas as pl
from jax.experimental.pallas import tpu as pltpu
from jax.experimental.pallas import tpu_sc as plsc

_NC = 2
_NS = 16
_RCH = 32


def kernel(inputs, embed_weight):
    B, S = inputs.shape
    E = embed_weight.shape[1]
    NW = _NC * _NS
    rows_per_w = S // NW
    n_chunks = rows_per_w // _RCH

    mesh = plsc.VectorSubcoreMesh(core_axis_name="c", subcore_axis_name="s")

    @functools.partial(
        pl.kernel,
        out_type=jax.ShapeDtypeStruct((B, S, E), embed_weight.dtype),
        mesh=mesh,
        scratch_types=[
            pltpu.VMEM((2, _RCH, E), embed_weight.dtype),
            pltpu.SemaphoreType.DMA((2,)),
            pltpu.SemaphoreType.DMA((2, B)),
        ],
    )
    def sc_copy(w_hbm, o_hbm, buf, in_sem, out_sem):
        wid = lax.axis_index("s") * _NC + lax.axis_index("c")
        base = wid * rows_per_w

        def in_cp(j, slot):
            return pltpu.make_async_copy(
                w_hbm.at[pl.ds(base + j * _RCH, _RCH), :],
                buf.at[slot],
                in_sem.at[slot],
            )

        def out_cp(j, b, slot):
            return pltpu.make_async_copy(
                buf.at[slot],
                o_hbm.at[b, pl.ds(base + j * _RCH, _RCH), :],
                out_sem.at[slot, b],
            )

        for j in range(n_chunks):
            slot = j % 2
            if j >= 2:
                for b in range(B):
                    out_cp(j - 2, b, slot).wait()
            in_cp(j, slot).start()
            in_cp(j, slot).wait()
            for b in range(B):
                out_cp(j, b, slot).start()
        for j in (n_chunks - 2, n_chunks - 1):
            for b in range(B):
                out_cp(j, b, j % 2).wait()

    return sc_copy(embed_weight)


# speedup vs baseline: 1.6525x; 1.6525x over previous
import jax
import jax.numpy as jnp
from jax.experimental import pallas as pl
from jax.experimental.pallas import tpu as pltpu

_NCH = 4


def kernel(inputs, embed_weight):
    B, S = inputs.shape
    E = embed_weight.shape[1]
    NCH = _NCH
    CH = S // NCH

    def body(w_hbm, o_hbm, buf, in_sem, out_sem):
        def in_cp(j):
            return pltpu.make_async_copy(
                w_hbm.at[pl.ds(j * CH, CH), :],
                buf.at[pl.ds(j * CH, CH), :],
                in_sem.at[j],
            )

        def out_cp(j, b):
            return pltpu.make_async_copy(
                buf.at[pl.ds(j * CH, CH), :],
                o_hbm.at[b, pl.ds(j * CH, CH), :],
                out_sem.at[j, b],
            )

        for j in range(NCH):
            in_cp(j).start()
        for j in range(NCH):
            in_cp(j).wait()
            for b in range(B):
                out_cp(j, b).start()
        for j in range(NCH):
            for b in range(B):
                out_cp(j, b).wait()

    out = pl.pallas_call(
        body,
        in_specs=[pl.BlockSpec(memory_space=pl.ANY)],
        out_specs=pl.BlockSpec(memory_space=pl.ANY),
        out_shape=jax.ShapeDtypeStruct((B, S, E), embed_weight.dtype),
        scratch_shapes=[
            pltpu.VMEM((S, E), embed_weight.dtype),
            pltpu.SemaphoreType.DMA((NCH,)),
            pltpu.SemaphoreType.DMA((NCH, B)),
        ],
    )(embed_weight)
    return out


# speedup vs baseline: 1.6527x; 1.0001x over previous
import jax
import jax.numpy as jnp
from jax.experimental import pallas as pl
from jax.experimental.pallas import tpu as pltpu

_NCH = 4


def kernel(inputs, embed_weight):
    B, S = inputs.shape
    E = embed_weight.shape[1]
    NCH = _NCH
    CH = S // NCH

    def body(w_hbm, o_hbm, buf, in_sem, out_sem):
        def in_cp(j):
            return pltpu.make_async_copy(
                w_hbm.at[pl.ds(j * CH, CH), :],
                buf.at[pl.ds(j * CH, CH), :],
                in_sem.at[j],
            )

        H = CH // 2

        def out_cp(j, b, h):
            return pltpu.make_async_copy(
                buf.at[pl.ds(j * CH + h * H, H), :],
                o_hbm.at[b, pl.ds(j * CH + h * H, H), :],
                out_sem.at[j, b],
            )

        for j in range(NCH):
            in_cp(j).start()
        for j in range(NCH):
            in_cp(j).wait()
            for b in range(B):
                for h in range(2):
                    out_cp(j, b, h).start()
        for j in range(NCH):
            for b in range(B):
                for h in range(2):
                    out_cp(j, b, h).wait()

    out = pl.pallas_call(
        body,
        in_specs=[pl.BlockSpec(memory_space=pl.ANY)],
        out_specs=pl.BlockSpec(memory_space=pl.ANY),
        out_shape=jax.ShapeDtypeStruct((B, S, E), embed_weight.dtype),
        scratch_shapes=[
            pltpu.VMEM((S, E), embed_weight.dtype),
            pltpu.SemaphoreType.DMA((NCH,)),
            pltpu.SemaphoreType.DMA((NCH, B)),
        ],
    )(embed_weight)
    return out
